# baseline (speedup 1.0000x reference)
import functools

import jax
import jax.numpy as jnp
from jax import lax
from jax.experimental import pallas as pl
from jax.experimental.pallas import tpu as pltpu
from jax.experimental.pallas import tpu_sc as plsc

E_EMB = 256
HIDDEN = 1024
LINK_OFF = 512
PRIOR_OFF = 544
TYPE_OFF = 576
STACK_ROWS = 640

SEQ = 50
SEQP = 56
GW = 128
BB = 16
TB = BB * SEQP


def _sc_entity_gather(table, ids_flat):
    tp = ids_flat.shape[0]
    idx2 = ids_flat.reshape(1, tp)
    mesh = plsc.VectorSubcoreMesh(core_axis_name="core",
                                  subcore_axis_name="subcore")

    @functools.partial(
        pl.kernel,
        out_type=jax.ShapeDtypeStruct((tp, E_EMB), jnp.float32),
        mesh=mesh)
    def gather_kernel(x_hbm, i_hbm, o_hbm):
        def body(i_vmem, o_vmem):
            pltpu.sync_copy(x_hbm.at[i_vmem.at[0]], o_vmem)

        pltpu.emit_pipeline(
            body,
            grid=(tp // GW,),
            in_specs=[pl.BlockSpec((1, GW), index_map=lambda i: (0, i))],
            out_specs=[pl.BlockSpec((GW, E_EMB), index_map=lambda i: (i, 0))],
            core_axis_name=("core", "subcore"),
            dimension_semantics=(pltpu.PARALLEL,),
        )(i_hbm, o_hbm)

    return gather_kernel(table, idx2)


def _tc_body(ent_ref, w_ref, tab_ref, g_ref, b_ref, idx_ref, o_ref):
    ent = ent_ref[...].astype(jnp.bfloat16)
    acc = jnp.dot(ent, w_ref[...], preferred_element_type=jnp.float32)

    idx4 = jnp.transpose(idx_ref[0], (1, 0))
    pos = idx4[:, 0:1]
    lnk = idx4[:, 1:2]
    pri = idx4[:, 2:3]
    typ = idx4[:, 3:4]
    io = lax.broadcasted_iota(jnp.int32, (TB, STACK_ROWS), 1)
    hot = ((io == pos) | (io == lnk + LINK_OFF)
           | (io == pri + PRIOR_OFF) | (io == typ + TYPE_OFF))
    acc = acc + jnp.dot(hot.astype(jnp.bfloat16), tab_ref[...],
                        preferred_element_type=jnp.float32)

    mu = jnp.mean(acc, axis=-1, keepdims=True)
    d = acc - mu
    var = jnp.mean(d * d, axis=-1, keepdims=True)
    res = d * lax.rsqrt(var + 1e-12) * g_ref[...] + b_ref[...]
    for j in range(BB):
        o_ref[j] = res[SEQP * j:SEQP * j + SEQ, :]


def kernel(entity_table, pos_table, type_table, link_table, prior_table,
           W_dense, ln_gamma, ln_beta, entity_ids, position_ids,
           token_type_ids, link_prob_ids, prior_prob_ids):
    b, l = entity_ids.shape
    tp = b * SEQP
    g1 = b // BB

    def padded(a):
        return jnp.pad(a.astype(jnp.int32), ((0, 0), (0, SEQP - l)))

    filler = (lax.broadcasted_iota(jnp.int32, (b, SEQP - l), 0) * (SEQP - l)
              + lax.broadcasted_iota(jnp.int32, (b, SEQP - l), 1))
    ids = jnp.concatenate(
        [entity_ids.astype(jnp.int32), filler], axis=1).reshape(tp)
    ent = _sc_entity_gather(entity_table, ids)

    stacked = jnp.concatenate(
        [pos_table, link_table, prior_table, type_table,
         jnp.zeros((STACK_ROWS - TYPE_OFF - 2, HIDDEN), jnp.float32)],
        axis=0).astype(jnp.bfloat16)
    w_bf = W_dense.astype(jnp.bfloat16)
    g2 = ln_gamma.reshape(1, HIDDEN)
    b2 = ln_beta.reshape(1, HIDDEN)

    idx4 = jnp.stack(
        [padded(position_ids), padded(link_prob_ids),
         padded(prior_prob_ids), padded(token_type_ids)],
        axis=0)
    idx4 = idx4.reshape(4, g1, TB).transpose(1, 0, 2)

    const = lambda i: (0, 0)
    out3 = pl.pallas_call(
        _tc_body,
        grid=(g1,),
        in_specs=[
            pl.BlockSpec((TB, E_EMB), lambda i: (i, 0)),
            pl.BlockSpec((E_EMB, HIDDEN), const),
            pl.BlockSpec((STACK_ROWS, HIDDEN), const),
            pl.BlockSpec((1, HIDDEN), const),
            pl.BlockSpec((1, HIDDEN), const),
            pl.BlockSpec((1, 4, TB), lambda i: (i, 0, 0)),
        ],
        out_specs=pl.BlockSpec((BB, SEQ, HIDDEN), lambda i: (i, 0, 0)),
        out_shape=jax.ShapeDtypeStruct((b, l, HIDDEN), jnp.float32),
        compiler_params=pltpu.CompilerParams(
            dimension_semantics=("parallel",)),
    )(ent, w_bf, stacked, g2, b2, idx4)

    return out3


# speedup vs baseline: 1.0451x; 1.0451x over previous
import functools

import jax
import jax.numpy as jnp
from jax import lax
from jax.experimental import pallas as pl
from jax.experimental.pallas import tpu as pltpu
from jax.experimental.pallas import tpu_sc as plsc

E_EMB = 256
HIDDEN = 1024
LINK_OFF = 512
PRIOR_OFF = 544
TYPE_OFF = 576
STACK_ROWS = 640

SEQ = 50
GW = 128
BB = 16
TB = BB * SEQ


def _sc_entity_gather(table, ids_flat):
    t = ids_flat.shape[0]
    idx2 = ids_flat.reshape(1, t)
    mesh = plsc.VectorSubcoreMesh(core_axis_name="core",
                                  subcore_axis_name="subcore")

    @functools.partial(
        pl.kernel,
        out_type=jax.ShapeDtypeStruct((t, E_EMB), jnp.float32),
        mesh=mesh)
    def gather_kernel(x_hbm, i_hbm, o_hbm):
        def body(i_vmem, o_vmem):
            pltpu.sync_copy(x_hbm.at[i_vmem.at[0]], o_vmem)

        pltpu.emit_pipeline(
            body,
            grid=(t // GW,),
            in_specs=[pl.BlockSpec((1, GW), index_map=lambda i: (0, i))],
            out_specs=[pl.BlockSpec((GW, E_EMB), index_map=lambda i: (i, 0))],
            core_axis_name=("core", "subcore"),
            dimension_semantics=(pltpu.PARALLEL,),
        )(i_hbm, o_hbm)

    return gather_kernel(table, idx2)


def _tc_body(ent_ref, w_ref, tab_ref, g_ref, b_ref, idx_ref, o_ref):
    ent = ent_ref[...].astype(jnp.bfloat16)
    acc = jnp.dot(ent, w_ref[...], preferred_element_type=jnp.float32)

    idx4 = jnp.transpose(idx_ref[0], (1, 0))
    pos = idx4[:, 0:1]
    lnk = idx4[:, 1:2]
    pri = idx4[:, 2:3]
    typ = idx4[:, 3:4]
    io = lax.broadcasted_iota(jnp.int32, (TB, STACK_ROWS), 1)
    hot = ((io == pos) | (io == lnk + LINK_OFF)
           | (io == pri + PRIOR_OFF) | (io == typ + TYPE_OFF))
    acc = acc + jnp.dot(hot.astype(jnp.bfloat16), tab_ref[...],
                        preferred_element_type=jnp.float32)

    mu = jnp.mean(acc, axis=-1, keepdims=True)
    d = acc - mu
    var = jnp.mean(d * d, axis=-1, keepdims=True)
    res = d * lax.rsqrt(var + 1e-12) * g_ref[...] + b_ref[...]
    for j in range(BB):
        o_ref[j] = res[SEQ * j:SEQ * j + SEQ, :]


def kernel(entity_table, pos_table, type_table, link_table, prior_table,
           W_dense, ln_gamma, ln_beta, entity_ids, position_ids,
           token_type_ids, link_prob_ids, prior_prob_ids):
    b, l = entity_ids.shape
    t = b * l
    grid = t // TB

    ids = entity_ids.reshape(t).astype(jnp.int32)
    ent = _sc_entity_gather(entity_table, ids)

    stacked = jnp.concatenate(
        [pos_table, link_table, prior_table, type_table,
         jnp.zeros((STACK_ROWS - TYPE_OFF - 2, HIDDEN), jnp.float32)],
        axis=0).astype(jnp.bfloat16)
    w_bf = W_dense.astype(jnp.bfloat16)
    g2 = ln_gamma.reshape(1, HIDDEN)
    b2 = ln_beta.reshape(1, HIDDEN)

    idx4 = jnp.stack(
        [position_ids.reshape(t), link_prob_ids.reshape(t),
         prior_prob_ids.reshape(t), token_type_ids.reshape(t)],
        axis=0).astype(jnp.int32)
    idx4 = idx4.reshape(4, grid, TB).transpose(1, 0, 2)

    const = lambda i: (0, 0)
    out3 = pl.pallas_call(
        _tc_body,
        grid=(grid,),
        in_specs=[
            pl.BlockSpec((TB, E_EMB), lambda i: (i, 0)),
            pl.BlockSpec((E_EMB, HIDDEN), const),
            pl.BlockSpec((STACK_ROWS, HIDDEN), const),
            pl.BlockSpec((1, HIDDEN), const),
            pl.BlockSpec((1, HIDDEN), const),
            pl.BlockSpec((1, 4, TB), lambda i: (i, 0, 0)),
        ],
        out_specs=pl.BlockSpec((BB, SEQ, HIDDEN), lambda i: (i, 0, 0)),
        out_shape=jax.ShapeDtypeStruct((b, l, HIDDEN), jnp.float32),
        compiler_params=pltpu.CompilerParams(
            dimension_semantics=("parallel",)),
    )(ent, w_bf, stacked, g2, b2, idx4)

    return out3
